# baseline (speedup 1.0000x reference)
import functools

import jax
import jax.numpy as jnp
from jax import lax
from jax.experimental import pallas as pl
from jax.experimental.pallas import tpu as pltpu
from jax.experimental.pallas import tpu_sc as plsc

_VOCAB = 100
_H = 128

_NC = 2
_NS = 16
_NW = _NC * _NS

_CHUNK = 128
_CPW = 25
_NPAD = _NW * _CPW * _CHUNK
_TOTAL_CHUNKS = _NW * _CPW


def _proj_body(emb_ref, w_ref, b_ref, out_ref):
    out_ref[:] = (
        lax.dot_general(
            emb_ref[:],
            w_ref[:],
            (((1,), (1,)), ((), ())),
            preferred_element_type=jnp.float32,
            precision=lax.Precision.HIGHEST,
        )
        + b_ref[:]
    )


def _project_table(embed_table, W, b):
    return pl.pallas_call(
        _proj_body,
        out_shape=jax.ShapeDtypeStruct((_VOCAB, _H), jnp.float32),
    )(embed_table, W, b.reshape(1, _H))


_mesh = plsc.VectorSubcoreMesh(core_axis_name="c", subcore_axis_name="s")


@functools.partial(
    pl.kernel,
    mesh=_mesh,
    out_type=jax.ShapeDtypeStruct((_TOTAL_CHUNKS, _CHUNK, _H), jnp.float32),
    scratch_types=[
        pltpu.VMEM((_CPW, _CHUNK), jnp.int32),
        pltpu.VMEM((_CHUNK, _H), jnp.float32),
        pltpu.SemaphoreType.DMA,
    ],
)
def _gather_kernel(proj_hbm, idx_hbm, out_hbm, idx_v, rows_v, sem):
    wid = lax.axis_index("s") * _NC + lax.axis_index("c")
    base = wid * _CPW
    pltpu.sync_copy(idx_hbm.at[wid], idx_v)

    def body(c, carry):
        pltpu.async_copy(proj_hbm.at[idx_v.at[c]], rows_v, sem).wait()
        pltpu.sync_copy(rows_v, out_hbm.at[base + c])
        return carry

    lax.fori_loop(0, _CPW, body, 0)


def kernel(atom_types, embed_table, W, b):
    n = atom_types.shape[0]
    proj = _project_table(embed_table, W, b)
    idx = jnp.pad(atom_types.astype(jnp.int32), (0, _NPAD - n))
    out = _gather_kernel(proj, idx.reshape(_NW, _CPW, _CHUNK))
    return out.reshape(_NPAD, _H)[:n]


# speedup vs baseline: 1.2725x; 1.2725x over previous
import functools

import jax
import jax.numpy as jnp
from jax import lax
from jax.experimental import pallas as pl
from jax.experimental.pallas import tpu as pltpu
from jax.experimental.pallas import tpu_sc as plsc

_VOCAB = 100
_H = 128

_NC = 2
_NS = 16
_NW = _NC * _NS

_CHUNK = 128
_CPW = 25
_NPAD = _NW * _CPW * _CHUNK
_TOTAL_CHUNKS = _NW * _CPW


def _proj_body(emb_ref, w_ref, b_ref, out_ref):
    out_ref[0] = (
        lax.dot_general(
            emb_ref[:],
            w_ref[:],
            (((1,), (1,)), ((), ())),
            preferred_element_type=jnp.float32,
            precision=lax.Precision.HIGHEST,
        )
        + b_ref[:]
    )


def _project_table(embed_table, W, b):
    return pl.pallas_call(
        _proj_body,
        grid=(_NW,),
        in_specs=[
            pl.BlockSpec((_VOCAB, _H), lambda i: (0, 0)),
            pl.BlockSpec((_H, _H), lambda i: (0, 0)),
            pl.BlockSpec((1, _H), lambda i: (0, 0)),
        ],
        out_specs=pl.BlockSpec((1, _VOCAB, _H), lambda i: (i, 0, 0)),
        out_shape=jax.ShapeDtypeStruct((_NW, _VOCAB, _H), jnp.float32),
    )(embed_table, W, b.reshape(1, _H))


_mesh = plsc.VectorSubcoreMesh(core_axis_name="c", subcore_axis_name="s")


@functools.partial(
    pl.kernel,
    mesh=_mesh,
    out_type=jax.ShapeDtypeStruct((_TOTAL_CHUNKS, _CHUNK, _H), jnp.float32),
    scratch_types=[
        pltpu.VMEM((_CPW, _CHUNK), jnp.int32),
        pltpu.VMEM((_CHUNK, _H), jnp.float32),
        pltpu.SemaphoreType.DMA,
    ],
)
def _gather_kernel(proj_hbm, idx_hbm, out_hbm, idx_v, rows_v, sem):
    wid = lax.axis_index("s") * _NC + lax.axis_index("c")
    base = wid * _CPW
    my_proj = proj_hbm.at[wid]
    pltpu.sync_copy(idx_hbm.at[wid], idx_v)

    def body(c, carry):
        pltpu.async_copy(my_proj.at[idx_v.at[c]], rows_v, sem).wait()
        pltpu.sync_copy(rows_v, out_hbm.at[base + c])
        return carry

    lax.fori_loop(0, _CPW, body, 0)


def kernel(atom_types, embed_table, W, b):
    n = atom_types.shape[0]
    proj = _project_table(embed_table, W, b)
    idx = jnp.pad(atom_types.astype(jnp.int32), (0, _NPAD - n))
    out = _gather_kernel(proj, idx.reshape(_NW, _CPW, _CHUNK))
    return out.reshape(_NPAD, _H)[:n]


# speedup vs baseline: 1.4275x; 1.1219x over previous
import functools

import jax
import jax.numpy as jnp
from jax import lax
from jax.experimental import pallas as pl
from jax.experimental.pallas import tpu as pltpu
from jax.experimental.pallas import tpu_sc as plsc

_VOCAB = 100
_H = 128

_NC = 2
_NS = 16
_NW = _NC * _NS

_CHUNK = 128
_CPW = 25
_NBUF = 4
_NPAD = _NW * _CPW * _CHUNK
_TOTAL_CHUNKS = _NW * _CPW


def _proj_body(emb_ref, w_ref, b_ref, out_ref):
    out_ref[0] = (
        lax.dot_general(
            emb_ref[:],
            w_ref[:],
            (((1,), (1,)), ((), ())),
            preferred_element_type=jnp.float32,
            precision=lax.Precision.HIGHEST,
        )
        + b_ref[:]
    )


def _project_table(embed_table, W, b):
    return pl.pallas_call(
        _proj_body,
        grid=(_NW,),
        in_specs=[
            pl.BlockSpec((_VOCAB, _H), lambda i: (0, 0)),
            pl.BlockSpec((_H, _H), lambda i: (0, 0)),
            pl.BlockSpec((1, _H), lambda i: (0, 0)),
        ],
        out_specs=pl.BlockSpec((1, _VOCAB, _H), lambda i: (i, 0, 0)),
        out_shape=jax.ShapeDtypeStruct((_NW, _VOCAB, _H), jnp.float32),
    )(embed_table, W, b.reshape(1, _H))


_mesh = plsc.VectorSubcoreMesh(core_axis_name="c", subcore_axis_name="s")


@functools.partial(
    pl.kernel,
    mesh=_mesh,
    out_type=jax.ShapeDtypeStruct((_TOTAL_CHUNKS, _CHUNK, _H), jnp.float32),
    scratch_types=[
        pltpu.VMEM((_CPW, _CHUNK), jnp.int32),
        pltpu.VMEM((_NBUF, _CHUNK, _H), jnp.float32),
        pltpu.SemaphoreType.DMA,
        pltpu.SemaphoreType.DMA,
        pltpu.SemaphoreType.DMA,
        pltpu.SemaphoreType.DMA,
        pltpu.SemaphoreType.DMA,
        pltpu.SemaphoreType.DMA,
        pltpu.SemaphoreType.DMA,
        pltpu.SemaphoreType.DMA,
    ],
)
def _gather_kernel(
    proj_hbm, idx_hbm, out_hbm, idx_v, rows_v, g0, g1, g2, g3, w0, w1, w2, w3
):
    gsem = (g0, g1, g2, g3)
    wsem = (w0, w1, w2, w3)
    wid = lax.axis_index("s") * _NC + lax.axis_index("c")
    base = wid * _CPW
    my_proj = proj_hbm.at[wid]
    pltpu.sync_copy(idx_hbm.at[wid], idx_v)

    def start_gather(c, j):
        pltpu.async_copy(my_proj.at[idx_v.at[c]], rows_v.at[j], gsem[j])

    def wait_gather(c, j):
        pltpu.make_async_copy(my_proj.at[idx_v.at[c]], rows_v.at[j], gsem[j]).wait()

    def start_write(c, j):
        pltpu.async_copy(rows_v.at[j], out_hbm.at[base + c], wsem[j])

    def wait_write(c, j):
        pltpu.make_async_copy(rows_v.at[j], out_hbm.at[base + c], wsem[j]).wait()

    for j in range(_NBUF - 1):
        start_gather(j, j)

    def group(g, carry):
        for j in range(_NBUF):
            c = g * _NBUF + j
            jn = (j + _NBUF - 1) % _NBUF
            wait_gather(c, j)
            start_write(c, j)
            n = c + _NBUF - 1

            @pl.when(n < _CPW)
            def _():
                @pl.when(c > 0)
                def _():
                    wait_write(c - 1, jn)

                start_gather(n, jn)

        return carry

    lax.fori_loop(0, _CPW // _NBUF, group, 0)

    for c in range(_CPW - _CPW % _NBUF, _CPW):
        wait_gather(c, c % _NBUF)
        start_write(c, c % _NBUF)

    for c in range(_CPW - _NBUF, _CPW):
        wait_write(c, c % _NBUF)


def kernel(atom_types, embed_table, W, b):
    n = atom_types.shape[0]
    proj = _project_table(embed_table, W, b)
    idx = jnp.pad(atom_types.astype(jnp.int32), (0, _NPAD - n))
    out = _gather_kernel(proj, idx.reshape(_NW, _CPW, _CHUNK))
    return out.reshape(_NPAD, _H)[:n]


# speedup vs baseline: 3.9383x; 2.7588x over previous
import functools

import jax
import jax.numpy as jnp
from jax import lax
from jax.experimental import pallas as pl
from jax.experimental.pallas import tpu as pltpu
from jax.experimental.pallas import tpu_sc as plsc

_VOCAB = 100
_H = 128

_NC = 2
_NS = 16
_NW = _NC * _NS

_CHUNK = 128
_CPW = 25
_NBUF = 4
_NPAD = _NW * _CPW * _CHUNK
_TOTAL_CHUNKS = _NW * _CPW
_N = 100000
_LAST_FULL = (_N - (_NW - 1) * _CPW * _CHUNK) // _CHUNK
_LAST_TAIL = _N - (_NW - 1) * _CPW * _CHUNK - _LAST_FULL * _CHUNK


def _proj_body(emb_ref, w_ref, b_ref, out_ref):
    out_ref[0] = (
        lax.dot_general(
            emb_ref[:],
            w_ref[:],
            (((1,), (1,)), ((), ())),
            preferred_element_type=jnp.float32,
            precision=lax.Precision.HIGHEST,
        )
        + b_ref[:]
    )


def _project_table(embed_table, W, b):
    return pl.pallas_call(
        _proj_body,
        grid=(_NW,),
        in_specs=[
            pl.BlockSpec((_VOCAB, _H), lambda i: (0, 0)),
            pl.BlockSpec((_H, _H), lambda i: (0, 0)),
            pl.BlockSpec((1, _H), lambda i: (0, 0)),
        ],
        out_specs=pl.BlockSpec((1, _VOCAB, _H), lambda i: (i, 0, 0)),
        out_shape=jax.ShapeDtypeStruct((_NW, _VOCAB, _H), jnp.float32),
    )(embed_table, W, b.reshape(1, _H))


_mesh = plsc.VectorSubcoreMesh(core_axis_name="c", subcore_axis_name="s")


@functools.partial(
    pl.kernel,
    mesh=_mesh,
    out_type=jax.ShapeDtypeStruct((_N, _H), jnp.float32),
    scratch_types=[
        pltpu.VMEM((_CPW, _CHUNK), jnp.int32),
        pltpu.VMEM((_NBUF, _CHUNK, _H), jnp.float32),
        pltpu.SemaphoreType.DMA,
        pltpu.SemaphoreType.DMA,
        pltpu.SemaphoreType.DMA,
        pltpu.SemaphoreType.DMA,
        pltpu.SemaphoreType.DMA,
        pltpu.SemaphoreType.DMA,
        pltpu.SemaphoreType.DMA,
        pltpu.SemaphoreType.DMA,
    ],
)
def _gather_kernel(
    proj_hbm, idx_hbm, out_hbm, idx_v, rows_v, g0, g1, g2, g3, w0, w1, w2, w3
):
    gsem = (g0, g1, g2, g3)
    wsem = (w0, w1, w2, w3)
    wid = lax.axis_index("s") * _NC + lax.axis_index("c")
    base = wid * _CPW
    my_proj = proj_hbm.at[wid]
    pltpu.sync_copy(idx_hbm.at[wid], idx_v)

    def start_gather(c, j):
        pltpu.async_copy(my_proj.at[idx_v.at[c]], rows_v.at[j], gsem[j])

    def wait_gather(c, j):
        pltpu.make_async_copy(my_proj.at[idx_v.at[c]], rows_v.at[j], gsem[j]).wait()

    def run_chunks(nfull, tail_rows):
        total = nfull + (1 if tail_rows else 0)

        def write_refs(c, j, rows):
            src = rows_v.at[j] if rows == _CHUNK else rows_v.at[j].at[pl.ds(0, rows)]
            dst = out_hbm.at[pl.ds((base + c) * _CHUNK, rows)]
            return src, dst

        def start_write(c, j, rows=_CHUNK):
            src, dst = write_refs(c, j, rows)
            pltpu.async_copy(src, dst, wsem[j])

        def wait_write(c, j, rows=_CHUNK):
            src, dst = write_refs(c, j, rows)
            pltpu.make_async_copy(src, dst, wsem[j]).wait()

        for j in range(min(_NBUF - 1, total)):
            start_gather(j, j)

        def group(g, carry):
            for j in range(_NBUF):
                c = g * _NBUF + j
                jn = (j + _NBUF - 1) % _NBUF
                wait_gather(c, j)
                start_write(c, j)
                n = c + _NBUF - 1

                @pl.when(n < total)
                def _():
                    @pl.when(c > 0)
                    def _():
                        wait_write(c - 1, jn)

                    start_gather(n, jn)

            return carry

        ngroups = total // _NBUF
        if total % _NBUF == 0 and tail_rows:
            ngroups -= 1
        lax.fori_loop(0, ngroups, group, 0)

        for c in range(ngroups * _NBUF, total):
            rows = tail_rows if (tail_rows and c == total - 1) else _CHUNK
            wait_gather(c, c % _NBUF)
            start_write(c, c % _NBUF, rows)

        for c in range(max(0, total - _NBUF), total):
            rows = tail_rows if (tail_rows and c == total - 1) else _CHUNK
            wait_write(c, c % _NBUF, rows)

    @pl.when(wid < _NW - 1)
    def _():
        run_chunks(_CPW, 0)

    @pl.when(wid == _NW - 1)
    def _():
        run_chunks(_LAST_FULL, _LAST_TAIL)


def kernel(atom_types, embed_table, W, b):
    n = atom_types.shape[0]
    proj = _project_table(embed_table, W, b)
    idx = jnp.pad(atom_types.astype(jnp.int32), (0, _NPAD - n))
    return _gather_kernel(proj, idx.reshape(_NW, _CPW, _CHUNK))


# speedup vs baseline: 7.5564x; 1.9187x over previous
import functools

import jax
import jax.numpy as jnp
from jax import lax
from jax.experimental import pallas as pl
from jax.experimental.pallas import tpu as pltpu
from jax.experimental.pallas import tpu_sc as plsc

_VOCAB = 100
_H = 128

_NC = 2
_NS = 16
_NW = _NC * _NS

_CHUNK = 128
_CPW = 25
_NBUF = 4
_NPAD = _NW * _CPW * _CHUNK
_TOTAL_CHUNKS = _NW * _CPW
_N = 100000
_LAST_FULL = (_N - (_NW - 1) * _CPW * _CHUNK) // _CHUNK
_LAST_TAIL = _N - (_NW - 1) * _CPW * _CHUNK - _LAST_FULL * _CHUNK


def _proj_body(emb_ref, w_ref, b_ref, out_ref):
    out_ref[:] = (
        lax.dot_general(
            emb_ref[:],
            w_ref[:],
            (((1,), (1,)), ((), ())),
            preferred_element_type=jnp.float32,
            precision=lax.Precision.HIGHEST,
        )
        + b_ref[:]
    )


def _project_table(embed_table, W, b):
    return pl.pallas_call(
        _proj_body,
        out_shape=jax.ShapeDtypeStruct((_VOCAB, _H), jnp.float32),
    )(embed_table, W, b.reshape(1, _H))


_mesh = plsc.VectorSubcoreMesh(core_axis_name="c", subcore_axis_name="s")


@functools.partial(
    pl.kernel,
    mesh=_mesh,
    out_type=jax.ShapeDtypeStruct((_N, _H), jnp.float32),
    scratch_types=[
        pltpu.VMEM_SHARED((_VOCAB, _H), jnp.float32),
        pltpu.VMEM((_CPW, _CHUNK), jnp.int32),
        pltpu.VMEM((_NBUF, _CHUNK, _H), jnp.float32),
        pltpu.SemaphoreType.DMA,
        pltpu.SemaphoreType.DMA,
        pltpu.SemaphoreType.DMA,
        pltpu.SemaphoreType.DMA,
        pltpu.SemaphoreType.DMA,
        pltpu.SemaphoreType.DMA,
        pltpu.SemaphoreType.DMA,
        pltpu.SemaphoreType.DMA,
    ],
)
def _gather_kernel(
    proj_hbm, idx_hbm, out_hbm, proj_v, idx_v, rows_v, g0, g1, g2, g3, w0, w1, w2, w3
):
    gsem = (g0, g1, g2, g3)
    wsem = (w0, w1, w2, w3)
    wid = lax.axis_index("s") * _NC + lax.axis_index("c")
    base = wid * _CPW
    @pl.when(lax.axis_index("s") == 0)
    def _():
        pltpu.sync_copy(proj_hbm, proj_v)

    plsc.subcore_barrier()
    my_proj = proj_v
    pltpu.sync_copy(idx_hbm.at[wid], idx_v)

    def start_gather(c, j):
        pltpu.async_copy(my_proj.at[idx_v.at[c]], rows_v.at[j], gsem[j])

    def wait_gather(c, j):
        pltpu.make_async_copy(my_proj.at[idx_v.at[c]], rows_v.at[j], gsem[j]).wait()

    def run_chunks(nfull, tail_rows):
        total = nfull + (1 if tail_rows else 0)

        def write_refs(c, j, rows):
            src = rows_v.at[j] if rows == _CHUNK else rows_v.at[j].at[pl.ds(0, rows)]
            dst = out_hbm.at[pl.ds((base + c) * _CHUNK, rows)]
            return src, dst

        def start_write(c, j, rows=_CHUNK):
            src, dst = write_refs(c, j, rows)
            pltpu.async_copy(src, dst, wsem[j])

        def wait_write(c, j, rows=_CHUNK):
            src, dst = write_refs(c, j, rows)
            pltpu.make_async_copy(src, dst, wsem[j]).wait()

        for j in range(min(_NBUF - 1, total)):
            start_gather(j, j)

        def group(g, carry):
            for j in range(_NBUF):
                c = g * _NBUF + j
                jn = (j + _NBUF - 1) % _NBUF
                wait_gather(c, j)
                start_write(c, j)
                n = c + _NBUF - 1

                @pl.when(n < total)
                def _():
                    @pl.when(c > 0)
                    def _():
                        wait_write(c - 1, jn)

                    start_gather(n, jn)

            return carry

        ngroups = total // _NBUF
        if total % _NBUF == 0 and tail_rows:
            ngroups -= 1
        lax.fori_loop(0, ngroups, group, 0)

        for c in range(ngroups * _NBUF, total):
            rows = tail_rows if (tail_rows and c == total - 1) else _CHUNK
            wait_gather(c, c % _NBUF)
            start_write(c, c % _NBUF, rows)

        for c in range(max(0, total - _NBUF), total):
            rows = tail_rows if (tail_rows and c == total - 1) else _CHUNK
            wait_write(c, c % _NBUF, rows)

    @pl.when(wid < _NW - 1)
    def _():
        run_chunks(_CPW, 0)

    @pl.when(wid == _NW - 1)
    def _():
        run_chunks(_LAST_FULL, _LAST_TAIL)


def kernel(atom_types, embed_table, W, b):
    n = atom_types.shape[0]
    proj = _project_table(embed_table, W, b)
    idx = jnp.pad(atom_types.astype(jnp.int32), (0, _NPAD - n))
    return _gather_kernel(proj, idx.reshape(_NW, _CPW, _CHUNK))
